# baseline (speedup 1.0000x reference)
import jax
import jax.numpy as jnp
from jax import lax
from jax.experimental import pallas as pl
from jax.experimental.pallas import tpu as pltpu, tpu_sc as plsc

T, D, FF, E, R = 2048, 1024, 2816, 8, 16
ER = E * R
TOP_K = 2
LORA_SCALE = 2.0
TB = 256

_NC, _NS, _L = 2, 16, 16
_NW = _NC * _NS
_TPW = T // _NW


def _sc_mask(gvt):
    mesh = plsc.VectorSubcoreMesh(core_axis_name="c", subcore_axis_name="s")

    @pl.kernel(
        mesh=mesh,
        out_type=jax.ShapeDtypeStruct((E, T), jnp.float32),
        scratch_types=[
            pltpu.VMEM((E, _TPW), jnp.float32),
            pltpu.VMEM((E, _TPW), jnp.float32),
        ],
    )
    def k(gvt_hbm, out_hbm, gv_v, mask_v):
        wid = lax.axis_index("s") * _NC + lax.axis_index("c")
        base = wid * _TPW
        for e in range(E):
            pltpu.sync_copy(gvt_hbm.at[e, pl.ds(base, _TPW)], gv_v.at[e])
        for c in range(0, _TPW, _L):
            keys = []
            for e in range(E):
                v = gv_v[e, pl.ds(c, _L)]
                ik = lax.bitcast_convert_type(v, jnp.int32)
                keys.append(jnp.where(ik < 0, ik ^ jnp.int32(0x7FFFFFFF), ik))
            for e in range(E):
                rank = jnp.zeros((_L,), jnp.int32)
                for j in range(E):
                    if j == e:
                        continue
                    beats = keys[j] > keys[e] if j > e else keys[j] >= keys[e]
                    rank = rank + jnp.where(beats, 1, 0)
                mask_v[e, pl.ds(c, _L)] = jnp.where(rank < TOP_K, 1.0, 0.0)
        for e in range(E):
            pltpu.sync_copy(mask_v.at[e], out_hbm.at[e, pl.ds(base, _TPW)])

    return k(gvt)


def _mlp_kernel(mt_ref, x_ref, wg_ref, wu_ref, wd_ref, ag_ref, bg_ref,
                au_ref, bu_ref, ad_ref, bd_ref, out_ref):
    f32 = jnp.float32
    bf16 = jnp.bfloat16
    mask = jnp.transpose(mt_ref[...])
    sel_r = jax.lax.broadcasted_iota(jnp.int32, (E, ER), 0)
    sel_c = jax.lax.broadcasted_iota(jnp.int32, (E, ER), 1)
    sel = (sel_r == sel_c // R).astype(f32)
    me = jnp.dot(mask, sel, preferred_element_type=f32)

    xb = x_ref[...].astype(bf16)
    mid_g = jnp.dot(xb, ag_ref[...].astype(bf16), preferred_element_type=f32)
    mid_u = jnp.dot(xb, au_ref[...].astype(bf16), preferred_element_type=f32)
    mid_g = (mid_g * me).astype(bf16)
    mid_u = (mid_u * me).astype(bf16)
    g = (jnp.dot(xb, wg_ref[...].astype(bf16), preferred_element_type=f32)
         + LORA_SCALE * jnp.dot(mid_g, bg_ref[...].astype(bf16),
                                preferred_element_type=f32))
    u = (jnp.dot(xb, wu_ref[...].astype(bf16), preferred_element_type=f32)
         + LORA_SCALE * jnp.dot(mid_u, bu_ref[...].astype(bf16),
                                preferred_element_type=f32))
    h = (g * jax.nn.sigmoid(g)) * u
    hb = h.astype(bf16)
    mid_d = jnp.dot(hb, ad_ref[...].astype(bf16), preferred_element_type=f32)
    mid_d = (mid_d * me).astype(bf16)
    out_ref[...] = (
        jnp.dot(hb, wd_ref[...].astype(bf16), preferred_element_type=f32)
        + LORA_SCALE * jnp.dot(mid_d, bd_ref[...].astype(bf16),
                               preferred_element_type=f32))


@jax.jit
def kernel(x, gate_values, W_gate, W_up, W_down, A_gate, B_gate, A_up, B_up,
           A_down, B_down):
    ag = A_gate.transpose(1, 0, 2).reshape(D, ER)
    au = A_up.transpose(1, 0, 2).reshape(D, ER)
    ad = A_down.transpose(1, 0, 2).reshape(FF, ER)
    bg = B_gate.reshape(ER, FF)
    bu = B_up.reshape(ER, FF)
    bd = B_down.reshape(ER, D)

    maskt = _sc_mask(gate_values.T)

    grid = (T // TB,)
    tok = lambda i: (i, 0)
    tokc = lambda i: (0, i)
    full = lambda i: (0, 0)
    out = pl.pallas_call(
        _mlp_kernel,
        grid=grid,
        in_specs=[
            pl.BlockSpec((E, TB), tokc),
            pl.BlockSpec((TB, D), tok),
            pl.BlockSpec((D, FF), full),
            pl.BlockSpec((D, FF), full),
            pl.BlockSpec((FF, D), full),
            pl.BlockSpec((D, ER), full),
            pl.BlockSpec((ER, FF), full),
            pl.BlockSpec((D, ER), full),
            pl.BlockSpec((ER, FF), full),
            pl.BlockSpec((FF, ER), full),
            pl.BlockSpec((ER, D), full),
        ],
        out_specs=pl.BlockSpec((TB, D), tok),
        out_shape=jax.ShapeDtypeStruct((T, D), jnp.float32),
    )(maskt, x, W_gate, W_up, W_down, ag, bg, au, bu, ad, bd)
    return out


# speedup vs baseline: 1.2357x; 1.2357x over previous
import jax
import jax.numpy as jnp
from jax.experimental import pallas as pl
from jax.experimental.pallas import tpu as pltpu

T, D, FF, E, R = 2048, 1024, 2816, 8, 16
ER = E * R
TOP_K = 2
LORA_SCALE = 2.0
TB = 256


def _mlp_kernel(gv_ref, x_ref, wg_ref, wu_ref, wd_ref, ag_ref, bg_ref,
                au_ref, bu_ref, ad_ref, bd_ref, out_ref, me_ref):
    f32 = jnp.float32
    bf16 = jnp.bfloat16
    i = pl.program_id(0)

    @pl.when(i == 0)
    def _():
        gvt = gv_ref[...]
        ik = jax.lax.bitcast_convert_type(gvt, jnp.int32)
        key = jnp.where(ik < 0, ik ^ jnp.int32(0x7FFFFFFF), ik)
        e_idx = jax.lax.broadcasted_iota(jnp.int32, (E, T), 0)
        rank = jnp.zeros((E, T), f32)
        for j in range(E):
            kj = jnp.broadcast_to(key[j:j + 1, :], (E, T))
            ge = jnp.where(kj >= key, 1.0, 0.0)
            gt = jnp.where(kj > key, 1.0, 0.0)
            rank = rank + jnp.where(e_idx > j, ge, gt)
        mask = jnp.transpose((rank < TOP_K).astype(f32))
        sel_r = jax.lax.broadcasted_iota(jnp.int32, (E, ER), 0)
        sel_c = jax.lax.broadcasted_iota(jnp.int32, (E, ER), 1)
        sel = (sel_r == sel_c // R).astype(f32)
        me_ref[...] = jnp.dot(mask, sel, preferred_element_type=f32)

    me = me_ref[pl.ds(i * TB, TB), :]

    xb = x_ref[...].astype(bf16)
    mid_g = jnp.dot(xb, ag_ref[...].astype(bf16), preferred_element_type=f32)
    mid_u = jnp.dot(xb, au_ref[...].astype(bf16), preferred_element_type=f32)
    mid_g = (mid_g * me).astype(bf16)
    mid_u = (mid_u * me).astype(bf16)
    g = (jnp.dot(xb, wg_ref[...].astype(bf16), preferred_element_type=f32)
         + LORA_SCALE * jnp.dot(mid_g, bg_ref[...].astype(bf16),
                                preferred_element_type=f32))
    u = (jnp.dot(xb, wu_ref[...].astype(bf16), preferred_element_type=f32)
         + LORA_SCALE * jnp.dot(mid_u, bu_ref[...].astype(bf16),
                                preferred_element_type=f32))
    h = (g * jax.nn.sigmoid(g)) * u
    hb = h.astype(bf16)
    mid_d = jnp.dot(hb, ad_ref[...].astype(bf16), preferred_element_type=f32)
    mid_d = (mid_d * me).astype(bf16)
    out_ref[...] = (
        jnp.dot(hb, wd_ref[...].astype(bf16), preferred_element_type=f32)
        + LORA_SCALE * jnp.dot(mid_d, bd_ref[...].astype(bf16),
                               preferred_element_type=f32))


@jax.jit
def kernel(x, gate_values, W_gate, W_up, W_down, A_gate, B_gate, A_up, B_up,
           A_down, B_down):
    ag = A_gate.transpose(1, 0, 2).reshape(D, ER)
    au = A_up.transpose(1, 0, 2).reshape(D, ER)
    ad = A_down.transpose(1, 0, 2).reshape(FF, ER)
    bg = B_gate.reshape(ER, FF)
    bu = B_up.reshape(ER, FF)
    bd = B_down.reshape(ER, D)

    gvt = gate_values.T

    grid = (T // TB,)
    tok = lambda i: (i, 0)
    tokc = lambda i: (0, i)
    full = lambda i: (0, 0)
    out = pl.pallas_call(
        _mlp_kernel,
        grid=grid,
        in_specs=[
            pl.BlockSpec((E, T), full),
            pl.BlockSpec((TB, D), tok),
            pl.BlockSpec((D, FF), full),
            pl.BlockSpec((D, FF), full),
            pl.BlockSpec((FF, D), full),
            pl.BlockSpec((D, ER), full),
            pl.BlockSpec((ER, FF), full),
            pl.BlockSpec((D, ER), full),
            pl.BlockSpec((ER, FF), full),
            pl.BlockSpec((FF, ER), full),
            pl.BlockSpec((ER, D), full),
        ],
        out_specs=pl.BlockSpec((TB, D), tok),
        out_shape=jax.ShapeDtypeStruct((T, D), jnp.float32),
        scratch_shapes=[pltpu.VMEM((T, ER), jnp.float32)],
    )(gvt, x, W_gate, W_up, W_down, ag, bg, au, bu, ad, bd)
    return out
